# baseline (speedup 1.0000x reference)
import functools

import jax
import jax.numpy as jnp
from jax import lax
from jax.experimental import pallas as pl
from jax.experimental.pallas import tpu as pltpu
from jax.experimental.pallas import tpu_sc as plsc

_D = 64
_NW = 32
_CHUNK = 800


def _gather_rows(table, idx):
    n = idx.shape[0]
    b_per_w = n // _NW
    n_chunks = b_per_w // _CHUNK
    mesh = plsc.VectorSubcoreMesh(core_axis_name="c", subcore_axis_name="s")

    @functools.partial(
        pl.kernel,
        out_type=jax.ShapeDtypeStruct((n, _D), jnp.float32),
        mesh=mesh,
        scratch_types=[
            pltpu.VMEM((_CHUNK,), jnp.int32),
            pltpu.VMEM((_CHUNK, _D), jnp.float32),
            pltpu.SemaphoreType.DMA,
        ],
        compiler_params=pltpu.CompilerParams(use_tc_tiling_on_sc=False),
    )
    def k(table_hbm, idx_hbm, out_hbm, idx_v, rows_v, sem):
        wid = lax.axis_index("s") * 2 + lax.axis_index("c")
        w_base = wid * b_per_w

        def body(c, _):
            base = pl.multiple_of(w_base + c * _CHUNK, 8)
            pltpu.sync_copy(idx_hbm.at[pl.ds(base, _CHUNK)], idx_v)
            pltpu.async_copy(table_hbm.at[idx_v], rows_v, sem).wait()
            pltpu.sync_copy(rows_v, out_hbm.at[pl.ds(base, _CHUNK)])
            return ()

        lax.fori_loop(0, n_chunks, body, (), unroll=False)

    return k(table, idx)


def kernel(x, embeddings):
    idx = x.reshape(-1).astype(jnp.int32)
    out = _gather_rows(embeddings, idx)
    return out.reshape(x.shape + (embeddings.shape[1],))


# speedup vs baseline: 1.0090x; 1.0090x over previous
import functools

import jax
import jax.numpy as jnp
from jax import lax
from jax.experimental import pallas as pl
from jax.experimental.pallas import tpu as pltpu
from jax.experimental.pallas import tpu_sc as plsc

_D = 64
_NW = 32
_CHUNK = 800
_NBUF = 2


def _gather_rows(table, idx3):
    n_chunks = idx3.shape[1]
    b_per_w = n_chunks * _CHUNK
    n = _NW * b_per_w
    mesh = plsc.VectorSubcoreMesh(core_axis_name="c", subcore_axis_name="s")

    @functools.partial(
        pl.kernel,
        out_type=jax.ShapeDtypeStruct((n, _D), jnp.float32),
        mesh=mesh,
        scratch_types=[
            pltpu.VMEM((n_chunks, _CHUNK), jnp.int32),
            pltpu.VMEM((_NBUF, _CHUNK, _D), jnp.float32),
            pltpu.SemaphoreType.DMA,
        ],
        compiler_params=pltpu.CompilerParams(use_tc_tiling_on_sc=False),
    )
    def k(table_hbm, idx_hbm, out_hbm, idx_v, rows_v, gsem):
        wid = lax.axis_index("s") * 2 + lax.axis_index("c")
        w_base = wid * b_per_w
        pltpu.sync_copy(idx_hbm.at[wid], idx_v)

        def start_gather(c, b):
            pltpu.async_copy(table_hbm.at[idx_v.at[c]], rows_v.at[b], gsem)

        def finish(c, b):
            pltpu.make_async_copy(
                table_hbm.at[idx_v.at[c]], rows_v.at[b], gsem
            ).wait()
            base = pl.multiple_of(w_base + c * _CHUNK, 8)
            pltpu.sync_copy(rows_v.at[b], out_hbm.at[pl.ds(base, _CHUNK)])

        for b in range(_NBUF):
            start_gather(b, b)

        def body(c, _):
            b = lax.rem(c, _NBUF)
            finish(c, b)
            start_gather(c + _NBUF, b)
            return ()

        lax.fori_loop(0, n_chunks - _NBUF, body, (), unroll=False)
        for t in range(n_chunks - _NBUF, n_chunks):
            finish(t, t % _NBUF)

    return k(table, idx3)


def kernel(x, embeddings):
    idx = x.reshape(-1).astype(jnp.int32)
    b_per_w = idx.shape[0] // _NW
    idx3 = idx.reshape(_NW, b_per_w // _CHUNK, _CHUNK)
    out = _gather_rows(embeddings, idx3)
    return out.reshape(x.shape + (embeddings.shape[1],))


# speedup vs baseline: 1.1285x; 1.1185x over previous
import functools

import jax
import jax.numpy as jnp
from jax import lax
from jax.experimental import pallas as pl
from jax.experimental.pallas import tpu as pltpu
from jax.experimental.pallas import tpu_sc as plsc

_D = 64
_DP = 128
_NW = 32
_CHUNK = 400
_NBUF = 2
_S = 50
_SP = 56


def _gather_rows(table, idx3):
    n_chunks = idx3.shape[1]
    b_per_w = n_chunks * _CHUNK
    n = _NW * b_per_w
    n_rows = n // _S
    mesh = plsc.VectorSubcoreMesh(core_axis_name="c", subcore_axis_name="s")

    @functools.partial(
        pl.kernel,
        out_type=jax.ShapeDtypeStruct((n_rows * _SP, _DP), jnp.float32),
        mesh=mesh,
        scratch_types=[
            pltpu.VMEM((n_chunks, _CHUNK), jnp.int32),
            pltpu.VMEM((_NBUF, _CHUNK, _D), jnp.float32),
            pltpu.SemaphoreType.DMA,
        ],
        compiler_params=pltpu.CompilerParams(use_tc_tiling_on_sc=False),
    )
    def k(table_hbm, idx_hbm, out_hbm, idx_v, rows_v, gsem):
        wid = lax.axis_index("s") * 2 + lax.axis_index("c")
        w_base = wid * b_per_w
        pltpu.sync_copy(idx_hbm.at[wid], idx_v)

        def start_gather(c, b):
            pltpu.async_copy(table_hbm.at[idx_v.at[c]], rows_v.at[b], gsem)

        def finish(c, b):
            pltpu.make_async_copy(
                table_hbm.at[idx_v.at[c]], rows_v.at[b], gsem
            ).wait()
            first = w_base + c * _CHUNK
            nblk = first // _S
            for g in range(_CHUNK // _S):
                base = pl.multiple_of((nblk + g) * _SP, 8)
                pltpu.sync_copy(
                    rows_v.at[b, pl.ds(g * _S, _S), :],
                    out_hbm.at[pl.ds(base, _S), pl.ds(0, _D)],
                )

        for b in range(_NBUF):
            start_gather(b, b)

        def body(c, _):
            b = lax.rem(c, _NBUF)
            finish(c, b)
            start_gather(c + _NBUF, b)
            return ()

        lax.fori_loop(0, n_chunks - _NBUF, body, (), unroll=False)
        for t in range(n_chunks - _NBUF, n_chunks):
            finish(t, t % _NBUF)

    return k(table, idx3)


def kernel(x, embeddings):
    idx = x.reshape(-1).astype(jnp.int32)
    b_per_w = idx.shape[0] // _NW
    idx3 = idx.reshape(_NW, b_per_w // _CHUNK, _CHUNK)
    outp = _gather_rows(embeddings, idx3)
    out3 = outp.reshape(x.shape[0], _SP, _DP)
    return out3[:, :_S, :_D]
